# Initial kernel scaffold; baseline (speedup 1.0000x reference)
import jax
import jax.numpy as jnp
from jax.experimental import pallas as pl


def kernel(inp, table):
    raise NotImplementedError("write your pallas kernel here")


# speedup vs baseline: 1.8731x; 1.8731x over previous
import functools

import jax
import jax.numpy as jnp
from jax import lax
from jax.experimental import pallas as pl
from jax.experimental.pallas import tpu as pltpu
from jax.experimental.pallas import tpu_sc as plsc

_NUM_CORES = 2
_NUM_SUBCORES = 16
_NW = _NUM_CORES * _NUM_SUBCORES

_SB = 128
_NBUF = 8


@functools.partial(jax.jit, static_argnums=(2, 3, 4))
def _gather(idx, table, n_streams, n_buf, sb):
    n_groups = n_streams // n_buf
    _, d = table.shape
    n = _NW * n_streams * sb

    mesh = plsc.VectorSubcoreMesh(
        core_axis_name="c", subcore_axis_name="s",
        num_cores=_NUM_CORES, num_subcores=_NUM_SUBCORES)

    @functools.partial(
        pl.kernel,
        mesh=mesh,
        out_type=jax.ShapeDtypeStruct((n, d), jnp.float32),
        scratch_types=[
            pltpu.VMEM((n_streams, sb), jnp.int32),
            pltpu.VMEM((n_buf, sb, d), jnp.float32),
            pltpu.SemaphoreType.DMA((n_buf,)),
            pltpu.SemaphoreType.DMA((n_buf,)),
        ],
        compiler_params=pltpu.CompilerParams(use_tc_tiling_on_sc=False),
    )
    def run(idx_hbm, table_hbm, out_hbm, idx_v, rows_v, gsem, osem):
        wid = lax.axis_index("s") * _NUM_CORES + lax.axis_index("c")
        row_base = wid * (n_streams * sb)

        pltpu.sync_copy(idx_hbm.at[wid], idx_v)

        def fire_gather(j, b):
            pltpu.async_copy(
                table_hbm.at[idx_v.at[j]], rows_v.at[b], gsem.at[b])

        def wait_gather(b):
            pltpu.make_async_copy(
                table_hbm.at[idx_v.at[0]], rows_v.at[b], gsem.at[b]).wait()

        def fire_out(j, b):
            pltpu.async_copy(
                rows_v.at[b], out_hbm.at[pl.ds(row_base + j * sb, sb)],
                osem.at[b])

        def wait_out(b):
            pltpu.make_async_copy(
                rows_v.at[b], out_hbm.at[pl.ds(row_base, sb)],
                osem.at[b]).wait()

        for b in range(n_buf):
            fire_gather(b, b)

        @pl.loop(0, n_groups - 1)
        def _group(g):
            j0 = g * n_buf
            for b in range(n_buf):
                wait_gather(b)
                fire_out(j0 + b, b)
            for b in range(n_buf):
                wait_out(b)
                fire_gather(j0 + n_buf + b, b)

        for b in range(n_buf):
            wait_gather(b)
            fire_out((n_groups - 1) * n_buf + b, b)
        for b in range(n_buf):
            wait_out(b)

    return run(idx, table)


def kernel(inp, table):
    batch, hist = inp.shape
    _, d = table.shape
    n = batch * hist
    assert n % (_NW * _SB) == 0
    n_streams = n // (_NW * _SB)
    assert n_streams % _NBUF == 0

    idx = inp.reshape(_NW, n_streams, _SB).astype(jnp.int32)
    out = _gather(idx, table, n_streams, _NBUF, _SB)
    return out.reshape(batch, hist, d)
